# baseline (speedup 1.0000x reference)
import functools

import jax
import jax.numpy as jnp
from jax import lax
from jax.experimental import pallas as pl
from jax.experimental.pallas import tpu as pltpu
from jax.experimental.pallas import tpu_sc as plsc

N = 10000
NPAD = 10240
E = 320000
IN_DIM = 1536
HID = 256
OUT = 128
B = 100
NTILES = 16
ROWS_PER_TILE = NPAD // NTILES
IDXC = 10
NBUF = 3

_MESH = plsc.VectorSubcoreMesh(
    core_axis_name="c", subcore_axis_name="s", num_cores=2, num_subcores=16
)


DEGW = 128


def _deg_body(dst_idx, zeros, ones, out, dst_buf, ones_buf, acc):
    cid = lax.axis_index("c")
    sid = lax.axis_index("s")
    base = sid * ROWS_PER_TILE
    pltpu.sync_copy(zeros.at[pl.ds(base, ROWS_PER_TILE)],
                    acc.at[pl.ds(base, ROWS_PER_TILE)])
    pltpu.sync_copy(ones, ones_buf)
    pltpu.sync_copy(dst_idx.at[cid, sid], dst_buf)
    plsc.subcore_barrier()

    nb = E // (2 * NTILES * B)

    @pl.loop(0, nb)
    def _(k):
        pltpu.sync_copy(ones_buf, acc.at[dst_buf.at[k]], add=True)

    plsc.subcore_barrier()
    pltpu.sync_copy(acc.at[pl.ds(base, ROWS_PER_TILE)],
                    out.at[cid, pl.ds(base, ROWS_PER_TILE)])


_deg_call = functools.partial(
    pl.kernel,
    _deg_body,
    out_type=jax.ShapeDtypeStruct((2, NPAD, DEGW), jnp.float32),
    mesh=_MESH,
    scratch_types=[
        pltpu.VMEM((E // (2 * NTILES * B), B), jnp.int32),
        pltpu.VMEM((B, DEGW), jnp.float32),
        pltpu.VMEM_SHARED((NPAD, DEGW), jnp.float32),
    ],
)()


def _make_agg(nb, idxc):
    nch = nb // idxc

    def body(table, src_idx, dst_idx, zeros, out, srcA, dstA, srcB, dstB,
             rows0, rows1, rows2, acc, g0, g1, g2, isem):
        rows = (rows0, rows1, rows2)
        gsem = (g0, g1, g2)
        cid = lax.axis_index("c")
        sid = lax.axis_index("s")
        base = sid * ROWS_PER_TILE
        pltpu.sync_copy(zeros.at[pl.ds(base, ROWS_PER_TILE)],
                        acc.at[pl.ds(base, ROWS_PER_TILE)])
        pltpu.sync_copy(src_idx.at[cid, sid, 0], srcA)
        pltpu.sync_copy(dst_idx.at[cid, sid, 0], dstA)
        plsc.subcore_barrier()

        def process(src_buf, dst_buf, ch_next, nsrc, ndst):
            @pl.when(ch_next < nch)
            def _():
                pltpu.async_copy(src_idx.at[cid, sid, ch_next], nsrc, isem)
                pltpu.async_copy(dst_idx.at[cid, sid, ch_next], ndst, isem)
            for b in range(NBUF):
                pltpu.async_copy(table.at[src_buf.at[b]], rows[b], gsem[b])
            for k in range(idxc):
                s = k % NBUF
                pltpu.make_async_copy(
                    table.at[src_buf.at[k]], rows[s], gsem[s]).wait()
                pltpu.sync_copy(rows[s], acc.at[dst_buf.at[k]], add=True)
                if k + NBUF < idxc:
                    pltpu.async_copy(table.at[src_buf.at[k + NBUF]], rows[s],
                                     gsem[s])

            @pl.when(ch_next < nch)
            def _():
                pltpu.make_async_copy(
                    src_idx.at[cid, sid, ch_next], nsrc, isem).wait()
                pltpu.make_async_copy(
                    dst_idx.at[cid, sid, ch_next], ndst, isem).wait()

        @pl.loop(0, nch // 2)
        def _(i):
            process(srcA, dstA, 2 * i + 1, srcB, dstB)
            process(srcB, dstB, 2 * i + 2, srcA, dstA)

        plsc.subcore_barrier()
        pltpu.sync_copy(acc.at[pl.ds(base, ROWS_PER_TILE)],
                        out.at[cid, pl.ds(base, ROWS_PER_TILE)])

    return functools.partial(
        pl.kernel,
        body,
        out_type=jax.ShapeDtypeStruct((2, NPAD, 128), jnp.float32),
        mesh=_MESH,
        scratch_types=[
            pltpu.VMEM((idxc, B), jnp.int32),
            pltpu.VMEM((idxc, B), jnp.int32),
            pltpu.VMEM((idxc, B), jnp.int32),
            pltpu.VMEM((idxc, B), jnp.int32),
            pltpu.VMEM((B, 128), jnp.float32),
            pltpu.VMEM((B, 128), jnp.float32),
            pltpu.VMEM((B, 128), jnp.float32),
            pltpu.VMEM_SHARED((NPAD, 128), jnp.float32),
            pltpu.SemaphoreType.DMA,
            pltpu.SemaphoreType.DMA,
            pltpu.SemaphoreType.DMA,
            pltpu.SemaphoreType.DMA,
        ],
    )()


IDXC2 = IDXC // 2
_agg1_call = _make_agg(E // (NTILES * B), IDXC)
_agg2_call = _make_agg(E // (2 * NTILES * B), IDXC2)


_RB = 1000
_GRID = N // _RB


def _mm1a_body(x_ref, w_ref, h_ref):
    h_ref[...] = jnp.dot(x_ref[...], w_ref[...],
                         preferred_element_type=jnp.float32)


def _mm1a_call(x, W1):
    return pl.pallas_call(
        _mm1a_body,
        grid=(_GRID,),
        in_specs=[
            pl.BlockSpec((_RB, IN_DIM), lambda i: (i, 0)),
            pl.BlockSpec((IN_DIM, HID), lambda i: (0, 0)),
        ],
        out_specs=pl.BlockSpec((_RB, HID), lambda i: (i, 0)),
        out_shape=jax.ShapeDtypeStruct((N, HID), jnp.float32),
    )(x, W1)


def _mm1b_body(h_ref, deg_ref, g_ref, dinv_ref):
    d = deg_ref[0][:, :1] + deg_ref[1][:, :1] + 1.0
    dinv = lax.rsqrt(d)
    g = h_ref[...] * dinv
    g_ref[0] = g[:, :128]
    g_ref[1] = g[:, 128:]
    dinv_ref[...] = jnp.broadcast_to(dinv, (_RB, 128))


def _mm1b_call(h, deg2):
    return pl.pallas_call(
        _mm1b_body,
        grid=(_GRID,),
        in_specs=[
            pl.BlockSpec((_RB, HID), lambda i: (i, 0)),
            pl.BlockSpec((2, _RB, DEGW), lambda i: (0, i, 0)),
        ],
        out_specs=[
            pl.BlockSpec((2, _RB, 128), lambda i: (0, i, 0)),
            pl.BlockSpec((_RB, 128), lambda i: (i, 0)),
        ],
        out_shape=[
            jax.ShapeDtypeStruct((2, N, 128), jnp.float32),
            jax.ShapeDtypeStruct((N, 128), jnp.float32),
        ],
    )(h, deg2)


def _mm2_body(acc_ref, g_ref, dinv_ref, b1_ref, w2_ref, o_ref):
    d = dinv_ref[...]
    a0 = (acc_ref[0] + g_ref[0]) * d + b1_ref[:, :128]
    a1 = (acc_ref[1] + g_ref[1]) * d + b1_ref[:, 128:]
    h1 = jnp.maximum(jnp.concatenate([a0, a1], axis=1), 0.0)
    o_ref[...] = jnp.dot(h1, w2_ref[...],
                         preferred_element_type=jnp.float32) * d


def _mm2_call(acc1, gsplit, dinv, b1, W2):
    return pl.pallas_call(
        _mm2_body,
        grid=(_GRID,),
        in_specs=[
            pl.BlockSpec((2, _RB, 128), lambda i: (0, i, 0)),
            pl.BlockSpec((2, _RB, 128), lambda i: (0, i, 0)),
            pl.BlockSpec((_RB, 128), lambda i: (i, 0)),
            pl.BlockSpec((1, HID), lambda i: (0, 0)),
            pl.BlockSpec((HID, OUT), lambda i: (0, 0)),
        ],
        out_specs=pl.BlockSpec((_RB, OUT), lambda i: (i, 0)),
        out_shape=jax.ShapeDtypeStruct((N, OUT), jnp.float32),
    )(acc1, gsplit, dinv, b1, W2)


def _fin_body(acc_ref, g2_ref, dinv_ref, b2_ref, o_ref):
    h2 = (acc_ref[0] + acc_ref[1] + g2_ref[...]) * dinv_ref[...] + b2_ref[...]
    nrm = jnp.sqrt(jnp.sum(h2 * h2, axis=1, keepdims=True))
    o_ref[...] = h2 / jnp.maximum(nrm, 1e-12)


def _fin_call(acc2, g2, dinv, b2):
    return pl.pallas_call(
        _fin_body,
        grid=(_GRID,),
        in_specs=[
            pl.BlockSpec((2, _RB, OUT), lambda i: (0, i, 0)),
            pl.BlockSpec((_RB, OUT), lambda i: (i, 0)),
            pl.BlockSpec((_RB, 128), lambda i: (i, 0)),
            pl.BlockSpec((1, OUT), lambda i: (0, 0)),
        ],
        out_specs=pl.BlockSpec((_RB, OUT), lambda i: (i, 0)),
        out_shape=jax.ShapeDtypeStruct((N, OUT), jnp.float32),
    )(acc2, g2, dinv, b2)


def kernel(x, edge_index, W1, b1, W2, b2):
    src = edge_index[0].astype(jnp.int32)
    dst = edge_index[1].astype(jnp.int32)

    offs = (jnp.arange(2, dtype=jnp.int32) * N)[:, None]
    src_l1 = (src[None, :] + offs).reshape(2, NTILES, -1, IDXC, B)
    dst_l1 = jnp.broadcast_to(dst[None, :], (2, E)).reshape(2, NTILES, -1,
                                                            IDXC, B)
    src_l2 = src.reshape(2, NTILES, -1, IDXC2, B)
    dst_l2 = dst.reshape(2, NTILES, -1, IDXC2, B)
    dst_deg = dst.reshape(2, NTILES, -1, B)

    zeros128 = jnp.zeros((NPAD, 128), jnp.float32)
    zeros_deg = jnp.zeros((NPAD, DEGW), jnp.float32)
    ones_deg = jnp.ones((B, DEGW), jnp.float32)

    deg2 = _deg_call(dst_deg, zeros_deg, ones_deg)
    h = _mm1a_call(x, W1)
    gsplit, dinv = _mm1b_call(h, deg2)
    acc1 = _agg1_call(gsplit.reshape(2 * N, 128), src_l1, dst_l1, zeros128)
    g2 = _mm2_call(acc1, gsplit, dinv, b1.reshape(1, HID), W2)
    acc2 = _agg2_call(g2, src_l2, dst_l2, zeros128)
    return _fin_call(acc2, g2, dinv, b2.reshape(1, OUT))


# speedup vs baseline: 1.0264x; 1.0264x over previous
import functools

import jax
import jax.numpy as jnp
from jax import lax
from jax.experimental import pallas as pl
from jax.experimental.pallas import tpu as pltpu
from jax.experimental.pallas import tpu_sc as plsc

N = 10000
NPAD = 10240
E = 320000
IN_DIM = 1536
HID = 256
OUT = 128
B = 100
NTILES = 16
ROWS_PER_TILE = NPAD // NTILES
IDXC = 20
NBUF = 3

_MESH = plsc.VectorSubcoreMesh(
    core_axis_name="c", subcore_axis_name="s", num_cores=2, num_subcores=16
)


DEGW = 128


def _deg_body(dst_idx, zeros, ones, out, dst_buf, ones_buf, acc):
    cid = lax.axis_index("c")
    sid = lax.axis_index("s")
    base = sid * ROWS_PER_TILE
    pltpu.sync_copy(zeros.at[pl.ds(base, ROWS_PER_TILE)],
                    acc.at[pl.ds(base, ROWS_PER_TILE)])
    pltpu.sync_copy(ones, ones_buf)
    pltpu.sync_copy(dst_idx.at[cid, sid], dst_buf)
    plsc.subcore_barrier()

    nb = E // (2 * NTILES * B)

    @pl.loop(0, nb)
    def _(k):
        pltpu.sync_copy(ones_buf, acc.at[dst_buf.at[k]], add=True)

    plsc.subcore_barrier()
    pltpu.sync_copy(acc.at[pl.ds(base, ROWS_PER_TILE)],
                    out.at[cid, pl.ds(base, ROWS_PER_TILE)])


_deg_call = functools.partial(
    pl.kernel,
    _deg_body,
    out_type=jax.ShapeDtypeStruct((2, NPAD, DEGW), jnp.float32),
    mesh=_MESH,
    scratch_types=[
        pltpu.VMEM((E // (2 * NTILES * B), B), jnp.int32),
        pltpu.VMEM((B, DEGW), jnp.float32),
        pltpu.VMEM_SHARED((NPAD, DEGW), jnp.float32),
    ],
)()


def _make_agg(nb, idxc):
    nch = nb // idxc

    def body(table, src_idx, dst_idx, zeros, out, src_buf, dst_buf,
             rows0, rows1, rows2, acc, g0, g1, g2):
        rows = (rows0, rows1, rows2)
        gsem = (g0, g1, g2)
        cid = lax.axis_index("c")
        sid = lax.axis_index("s")
        base = sid * ROWS_PER_TILE
        pltpu.sync_copy(zeros.at[pl.ds(base, ROWS_PER_TILE)],
                        acc.at[pl.ds(base, ROWS_PER_TILE)])
        plsc.subcore_barrier()

        @pl.loop(0, nch)
        def _(ch):
            pltpu.sync_copy(src_idx.at[cid, sid, ch], src_buf)
            pltpu.sync_copy(dst_idx.at[cid, sid, ch], dst_buf)
            for b in range(NBUF):
                pltpu.async_copy(table.at[src_buf.at[b]], rows[b], gsem[b])
            for k in range(idxc):
                s = k % NBUF
                pltpu.make_async_copy(
                    table.at[src_buf.at[k]], rows[s], gsem[s]).wait()
                pltpu.sync_copy(rows[s], acc.at[dst_buf.at[k]], add=True)
                if k + NBUF < idxc:
                    pltpu.async_copy(table.at[src_buf.at[k + NBUF]], rows[s],
                                     gsem[s])

        plsc.subcore_barrier()
        pltpu.sync_copy(acc.at[pl.ds(base, ROWS_PER_TILE)],
                        out.at[cid, pl.ds(base, ROWS_PER_TILE)])

    return functools.partial(
        pl.kernel,
        body,
        out_type=jax.ShapeDtypeStruct((2, NPAD, 128), jnp.float32),
        mesh=_MESH,
        scratch_types=[
            pltpu.VMEM((idxc, B), jnp.int32),
            pltpu.VMEM((idxc, B), jnp.int32),
            pltpu.VMEM((B, 128), jnp.float32),
            pltpu.VMEM((B, 128), jnp.float32),
            pltpu.VMEM((B, 128), jnp.float32),
            pltpu.VMEM_SHARED((NPAD, 128), jnp.float32),
            pltpu.SemaphoreType.DMA,
            pltpu.SemaphoreType.DMA,
            pltpu.SemaphoreType.DMA,
        ],
    )()


IDXC2 = IDXC // 2
_agg1_call = _make_agg(E // (NTILES * B), IDXC)
_agg2_call = _make_agg(E // (2 * NTILES * B), IDXC2)


_RB = 1000
_GRID = N // _RB


def _mm1a_body(x_ref, w_ref, h_ref):
    h_ref[...] = jnp.dot(x_ref[...], w_ref[...],
                         preferred_element_type=jnp.float32)


def _mm1a_call(x, W1):
    return pl.pallas_call(
        _mm1a_body,
        grid=(_GRID,),
        in_specs=[
            pl.BlockSpec((_RB, IN_DIM), lambda i: (i, 0)),
            pl.BlockSpec((IN_DIM, HID), lambda i: (0, 0)),
        ],
        out_specs=pl.BlockSpec((_RB, HID), lambda i: (i, 0)),
        out_shape=jax.ShapeDtypeStruct((N, HID), jnp.float32),
    )(x, W1)


def _mm1b_body(h_ref, deg_ref, g_ref, dinv_ref):
    d = deg_ref[0][:, :1] + deg_ref[1][:, :1] + 1.0
    dinv = lax.rsqrt(d)
    g = h_ref[...] * dinv
    g_ref[0] = g[:, :128]
    g_ref[1] = g[:, 128:]
    dinv_ref[...] = jnp.broadcast_to(dinv, (_RB, 128))


def _mm1b_call(h, deg2):
    return pl.pallas_call(
        _mm1b_body,
        grid=(_GRID,),
        in_specs=[
            pl.BlockSpec((_RB, HID), lambda i: (i, 0)),
            pl.BlockSpec((2, _RB, DEGW), lambda i: (0, i, 0)),
        ],
        out_specs=[
            pl.BlockSpec((2, _RB, 128), lambda i: (0, i, 0)),
            pl.BlockSpec((_RB, 128), lambda i: (i, 0)),
        ],
        out_shape=[
            jax.ShapeDtypeStruct((2, N, 128), jnp.float32),
            jax.ShapeDtypeStruct((N, 128), jnp.float32),
        ],
    )(h, deg2)


def _mm2_body(acc_ref, g_ref, dinv_ref, b1_ref, w2_ref, o_ref):
    d = dinv_ref[...]
    a0 = (acc_ref[0] + g_ref[0]) * d + b1_ref[:, :128]
    a1 = (acc_ref[1] + g_ref[1]) * d + b1_ref[:, 128:]
    h1 = jnp.maximum(jnp.concatenate([a0, a1], axis=1), 0.0)
    o_ref[...] = jnp.dot(h1, w2_ref[...],
                         preferred_element_type=jnp.float32) * d


def _mm2_call(acc1, gsplit, dinv, b1, W2):
    return pl.pallas_call(
        _mm2_body,
        grid=(_GRID,),
        in_specs=[
            pl.BlockSpec((2, _RB, 128), lambda i: (0, i, 0)),
            pl.BlockSpec((2, _RB, 128), lambda i: (0, i, 0)),
            pl.BlockSpec((_RB, 128), lambda i: (i, 0)),
            pl.BlockSpec((1, HID), lambda i: (0, 0)),
            pl.BlockSpec((HID, OUT), lambda i: (0, 0)),
        ],
        out_specs=pl.BlockSpec((_RB, OUT), lambda i: (i, 0)),
        out_shape=jax.ShapeDtypeStruct((N, OUT), jnp.float32),
    )(acc1, gsplit, dinv, b1, W2)


def _fin_body(acc_ref, g2_ref, dinv_ref, b2_ref, o_ref):
    h2 = (acc_ref[0] + acc_ref[1] + g2_ref[...]) * dinv_ref[...] + b2_ref[...]
    nrm = jnp.sqrt(jnp.sum(h2 * h2, axis=1, keepdims=True))
    o_ref[...] = h2 / jnp.maximum(nrm, 1e-12)


def _fin_call(acc2, g2, dinv, b2):
    return pl.pallas_call(
        _fin_body,
        grid=(_GRID,),
        in_specs=[
            pl.BlockSpec((2, _RB, OUT), lambda i: (0, i, 0)),
            pl.BlockSpec((_RB, OUT), lambda i: (i, 0)),
            pl.BlockSpec((_RB, 128), lambda i: (i, 0)),
            pl.BlockSpec((1, OUT), lambda i: (0, 0)),
        ],
        out_specs=pl.BlockSpec((_RB, OUT), lambda i: (i, 0)),
        out_shape=jax.ShapeDtypeStruct((N, OUT), jnp.float32),
    )(acc2, g2, dinv, b2)


def kernel(x, edge_index, W1, b1, W2, b2):
    src = edge_index[0].astype(jnp.int32)
    dst = edge_index[1].astype(jnp.int32)

    offs = (jnp.arange(2, dtype=jnp.int32) * N)[:, None]
    src_l1 = (src[None, :] + offs).reshape(2, NTILES, -1, IDXC, B)
    dst_l1 = jnp.broadcast_to(dst[None, :], (2, E)).reshape(2, NTILES, -1,
                                                            IDXC, B)
    src_l2 = src.reshape(2, NTILES, -1, IDXC2, B)
    dst_l2 = dst.reshape(2, NTILES, -1, IDXC2, B)
    dst_deg = dst.reshape(2, NTILES, -1, B)

    zeros128 = jnp.zeros((NPAD, 128), jnp.float32)
    zeros_deg = jnp.zeros((NPAD, DEGW), jnp.float32)
    ones_deg = jnp.ones((B, DEGW), jnp.float32)

    deg2 = _deg_call(dst_deg, zeros_deg, ones_deg)
    h = _mm1a_call(x, W1)
    gsplit, dinv = _mm1b_call(h, deg2)
    acc1 = _agg1_call(gsplit.reshape(2 * N, 128), src_l1, dst_l1, zeros128)
    g2 = _mm2_call(acc1, gsplit, dinv, b1.reshape(1, HID), W2)
    acc2 = _agg2_call(g2, src_l2, dst_l2, zeros128)
    return _fin_call(acc2, g2, dinv, b2.reshape(1, OUT))


# speedup vs baseline: 1.0718x; 1.0442x over previous
import functools

import jax
import jax.numpy as jnp
from jax import lax
from jax.experimental import pallas as pl
from jax.experimental.pallas import tpu as pltpu
from jax.experimental.pallas import tpu_sc as plsc

N = 10000
NPAD = 10240
E = 320000
IN_DIM = 1536
HID = 256
OUT = 128
B = 100
NTILES = 16
ROWS_PER_TILE = NPAD // NTILES
IDXC = 20
NBUF = 3

_MESH = plsc.VectorSubcoreMesh(
    core_axis_name="c", subcore_axis_name="s", num_cores=2, num_subcores=16
)


DEGW = 128


def _deg_body(dst_idx, zeros, ones, out, dst_buf, ones_buf, acc):
    cid = lax.axis_index("c")
    sid = lax.axis_index("s")
    base = sid * ROWS_PER_TILE
    pltpu.sync_copy(zeros.at[pl.ds(base, ROWS_PER_TILE)],
                    acc.at[pl.ds(base, ROWS_PER_TILE)])
    pltpu.sync_copy(ones, ones_buf)
    pltpu.sync_copy(dst_idx.at[cid, sid], dst_buf)
    plsc.subcore_barrier()

    nb = E // (2 * NTILES * B)

    @pl.loop(0, nb)
    def _(k):
        pltpu.sync_copy(ones_buf, acc.at[dst_buf.at[k]], add=True)

    plsc.subcore_barrier()
    pltpu.sync_copy(acc.at[pl.ds(base, ROWS_PER_TILE)],
                    out.at[cid, pl.ds(base, ROWS_PER_TILE)])


_deg_call = functools.partial(
    pl.kernel,
    _deg_body,
    out_type=jax.ShapeDtypeStruct((2, NPAD, DEGW), jnp.float32),
    mesh=_MESH,
    scratch_types=[
        pltpu.VMEM((E // (2 * NTILES * B), B), jnp.int32),
        pltpu.VMEM((B, DEGW), jnp.float32),
        pltpu.VMEM_SHARED((NPAD, DEGW), jnp.float32),
    ],
)()


def _make_agg(nb, idxc, bsz):
    nch = nb // idxc

    def body(table, src_idx, dst_idx, zeros, out, srcA, dstA, srcB, dstB,
             rows0, rows1, rows2, acc, g0, g1, g2, isem):
        rows = (rows0, rows1, rows2)
        gsem = (g0, g1, g2)
        cid = lax.axis_index("c")
        sid = lax.axis_index("s")
        base = sid * ROWS_PER_TILE
        pltpu.sync_copy(zeros.at[pl.ds(base, ROWS_PER_TILE)],
                        acc.at[pl.ds(base, ROWS_PER_TILE)])
        pltpu.sync_copy(src_idx.at[cid, sid, 0], srcA)
        pltpu.sync_copy(dst_idx.at[cid, sid, 0], dstA)
        plsc.subcore_barrier()

        def process(src_buf, dst_buf, ch_next, nsrc, ndst):
            @pl.when(ch_next < nch)
            def _():
                pltpu.async_copy(src_idx.at[cid, sid, ch_next], nsrc, isem)
                pltpu.async_copy(dst_idx.at[cid, sid, ch_next], ndst, isem)
            for b in range(NBUF):
                pltpu.async_copy(table.at[src_buf.at[b]], rows[b], gsem[b])
            for k in range(idxc):
                s = k % NBUF
                pltpu.make_async_copy(
                    table.at[src_buf.at[k]], rows[s], gsem[s]).wait()
                pltpu.sync_copy(rows[s], acc.at[dst_buf.at[k]], add=True)
                if k + NBUF < idxc:
                    pltpu.async_copy(table.at[src_buf.at[k + NBUF]], rows[s],
                                     gsem[s])

            @pl.when(ch_next < nch)
            def _():
                pltpu.make_async_copy(
                    src_idx.at[cid, sid, ch_next], nsrc, isem).wait()
                pltpu.make_async_copy(
                    dst_idx.at[cid, sid, ch_next], ndst, isem).wait()

        @pl.loop(0, nch // 2)
        def _(i):
            process(srcA, dstA, 2 * i + 1, srcB, dstB)
            process(srcB, dstB, 2 * i + 2, srcA, dstA)

        plsc.subcore_barrier()
        pltpu.sync_copy(acc.at[pl.ds(base, ROWS_PER_TILE)],
                        out.at[cid, pl.ds(base, ROWS_PER_TILE)])

    return functools.partial(
        pl.kernel,
        body,
        out_type=jax.ShapeDtypeStruct((2, NPAD, 128), jnp.float32),
        mesh=_MESH,
        scratch_types=[
            pltpu.VMEM((idxc, bsz), jnp.int32),
            pltpu.VMEM((idxc, bsz), jnp.int32),
            pltpu.VMEM((idxc, bsz), jnp.int32),
            pltpu.VMEM((idxc, bsz), jnp.int32),
            pltpu.VMEM((bsz, 128), jnp.float32),
            pltpu.VMEM((bsz, 128), jnp.float32),
            pltpu.VMEM((bsz, 128), jnp.float32),
            pltpu.VMEM_SHARED((NPAD, 128), jnp.float32),
            pltpu.SemaphoreType.DMA,
            pltpu.SemaphoreType.DMA,
            pltpu.SemaphoreType.DMA,
            pltpu.SemaphoreType.DMA,
        ],
    )()


B1 = 80
IDXC1 = 25
IDXC2 = IDXC // 2
_agg1_call = _make_agg(E // (NTILES * B1), IDXC1, B1)
_agg2_call = _make_agg(E // (2 * NTILES * B), IDXC2, B)


_RB = 1000
_GRID = N // _RB


def _mm1a_body(x_ref, w_ref, h_ref):
    h_ref[...] = jnp.dot(x_ref[...], w_ref[...],
                         preferred_element_type=jnp.float32)


def _mm1a_call(x, W1):
    return pl.pallas_call(
        _mm1a_body,
        grid=(_GRID,),
        in_specs=[
            pl.BlockSpec((_RB, IN_DIM), lambda i: (i, 0)),
            pl.BlockSpec((IN_DIM, HID), lambda i: (0, 0)),
        ],
        out_specs=pl.BlockSpec((_RB, HID), lambda i: (i, 0)),
        out_shape=jax.ShapeDtypeStruct((N, HID), jnp.float32),
    )(x, W1)


def _mm1b_body(h_ref, deg_ref, g_ref, dinv_ref):
    d = deg_ref[0][:, :1] + deg_ref[1][:, :1] + 1.0
    dinv = lax.rsqrt(d)
    g = h_ref[...] * dinv
    g_ref[0] = g[:, :128]
    g_ref[1] = g[:, 128:]
    dinv_ref[...] = jnp.broadcast_to(dinv, (_RB, 128))


def _mm1b_call(h, deg2):
    return pl.pallas_call(
        _mm1b_body,
        grid=(_GRID,),
        in_specs=[
            pl.BlockSpec((_RB, HID), lambda i: (i, 0)),
            pl.BlockSpec((2, _RB, DEGW), lambda i: (0, i, 0)),
        ],
        out_specs=[
            pl.BlockSpec((2, _RB, 128), lambda i: (0, i, 0)),
            pl.BlockSpec((_RB, 128), lambda i: (i, 0)),
        ],
        out_shape=[
            jax.ShapeDtypeStruct((2, N, 128), jnp.float32),
            jax.ShapeDtypeStruct((N, 128), jnp.float32),
        ],
    )(h, deg2)


def _mm2_body(acc_ref, g_ref, dinv_ref, b1_ref, w2_ref, o_ref):
    d = dinv_ref[...]
    a0 = (acc_ref[0] + g_ref[0]) * d + b1_ref[:, :128]
    a1 = (acc_ref[1] + g_ref[1]) * d + b1_ref[:, 128:]
    h1 = jnp.maximum(jnp.concatenate([a0, a1], axis=1), 0.0)
    o_ref[...] = jnp.dot(h1, w2_ref[...],
                         preferred_element_type=jnp.float32) * d


def _mm2_call(acc1, gsplit, dinv, b1, W2):
    return pl.pallas_call(
        _mm2_body,
        grid=(_GRID,),
        in_specs=[
            pl.BlockSpec((2, _RB, 128), lambda i: (0, i, 0)),
            pl.BlockSpec((2, _RB, 128), lambda i: (0, i, 0)),
            pl.BlockSpec((_RB, 128), lambda i: (i, 0)),
            pl.BlockSpec((1, HID), lambda i: (0, 0)),
            pl.BlockSpec((HID, OUT), lambda i: (0, 0)),
        ],
        out_specs=pl.BlockSpec((_RB, OUT), lambda i: (i, 0)),
        out_shape=jax.ShapeDtypeStruct((N, OUT), jnp.float32),
    )(acc1, gsplit, dinv, b1, W2)


def _fin_body(acc_ref, g2_ref, dinv_ref, b2_ref, o_ref):
    h2 = (acc_ref[0] + acc_ref[1] + g2_ref[...]) * dinv_ref[...] + b2_ref[...]
    nrm = jnp.sqrt(jnp.sum(h2 * h2, axis=1, keepdims=True))
    o_ref[...] = h2 / jnp.maximum(nrm, 1e-12)


def _fin_call(acc2, g2, dinv, b2):
    return pl.pallas_call(
        _fin_body,
        grid=(_GRID,),
        in_specs=[
            pl.BlockSpec((2, _RB, OUT), lambda i: (0, i, 0)),
            pl.BlockSpec((_RB, OUT), lambda i: (i, 0)),
            pl.BlockSpec((_RB, 128), lambda i: (i, 0)),
            pl.BlockSpec((1, OUT), lambda i: (0, 0)),
        ],
        out_specs=pl.BlockSpec((_RB, OUT), lambda i: (i, 0)),
        out_shape=jax.ShapeDtypeStruct((N, OUT), jnp.float32),
    )(acc2, g2, dinv, b2)


def kernel(x, edge_index, W1, b1, W2, b2):
    src = edge_index[0].astype(jnp.int32)
    dst = edge_index[1].astype(jnp.int32)

    offs = (jnp.arange(2, dtype=jnp.int32) * N)[:, None]
    src_l1 = (src[None, :] + offs).reshape(2, NTILES, -1, IDXC1, B1)
    dst_l1 = jnp.broadcast_to(dst[None, :], (2, E)).reshape(2, NTILES, -1,
                                                            IDXC1, B1)
    src_l2 = src.reshape(2, NTILES, -1, IDXC2, B)
    dst_l2 = dst.reshape(2, NTILES, -1, IDXC2, B)
    dst_deg = dst.reshape(2, NTILES, -1, B)

    zeros128 = jnp.zeros((NPAD, 128), jnp.float32)
    zeros_deg = jnp.zeros((NPAD, DEGW), jnp.float32)
    ones_deg = jnp.ones((B, DEGW), jnp.float32)

    deg2 = _deg_call(dst_deg, zeros_deg, ones_deg)
    h = _mm1a_call(x, W1)
    gsplit, dinv = _mm1b_call(h, deg2)
    acc1 = _agg1_call(gsplit.reshape(2 * N, 128), src_l1, dst_l1, zeros128)
    g2 = _mm2_call(acc1, gsplit, dinv, b1.reshape(1, HID), W2)
    acc2 = _agg2_call(g2, src_l2, dst_l2, zeros128)
    return _fin_call(acc2, g2, dinv, b2.reshape(1, OUT))
